# baseline (speedup 1.0000x reference)
import functools

import jax
import jax.numpy as jnp
from jax import lax
from jax.experimental import pallas as pl
from jax.experimental.pallas import tpu as pltpu
from jax.experimental.pallas import tpu_sc as plsc

N = 10000
E = 320000
NFEAT = 128
NHID = 128
NCLASS = 64

NC, NS = 2, 16
NW = NC * NS
EPW = E // NW
K = 80
C = EPW // K
NACC = 10240
RPT = NACC // NS

_MESH = plsc.VectorSubcoreMesh(
    core_axis_name="c", subcore_axis_name="s", num_cores=NC, num_subcores=NS)


NBUF = 3
REM = (C - NBUF) % NBUF


def _make_segsum(D):
    @functools.partial(
        pl.kernel,
        out_type=jax.ShapeDtypeStruct((NC, NACC, D), jnp.float32),
        mesh=_MESH,
        scratch_types=[
            pltpu.VMEM((C * K,), jnp.int32),
        ] + [pltpu.VMEM((K, D), jnp.float32)] * NBUF
          + [pltpu.VMEM((1, K), jnp.int32)] * NBUF + [
            pltpu.VMEM_SHARED((NACC, D), jnp.float32),
        ] + [pltpu.SemaphoreType.DMA] * (2 * NBUF),
    )
    def segsum(table, srcw, dstw, zrows, out, idxs_v, *rest):
        rows = rest[:NBUF]
        dbuf = rest[NBUF:2 * NBUF]
        acc = rest[2 * NBUF]
        gsems = rest[2 * NBUF + 1:3 * NBUF + 1]
        dsems = rest[3 * NBUF + 1:]
        cid = lax.axis_index("c")
        sid = lax.axis_index("s")
        wid = sid * NC + cid
        pltpu.sync_copy(zrows, acc.at[pl.ds(sid * RPT, RPT)])
        pltpu.sync_copy(srcw.at[wid], idxs_v)
        plsc.subcore_barrier()

        def sidx(ci):
            return idxs_v.at[pl.ds(ci * K, K)]

        def fire_gather(ci, b):
            pltpu.async_copy(table.at[sidx(ci)], rows[b], gsems[b])

        def fire_didx(ci, b):
            pltpu.async_copy(dstw.at[wid * C + ci], dbuf[b], dsems[b])

        for ci in range(REM):
            pltpu.async_copy(dstw.at[wid * C + ci], dbuf[0], dsems[0])
            pltpu.async_copy(table.at[sidx(ci)], rows[0], gsems[0]).wait()
            pltpu.make_async_copy(
                dstw.at[wid * C + ci], dbuf[0], dsems[0]).wait()
            pltpu.sync_copy(rows[0], acc.at[dbuf[0].at[0]], add=True)

        for b in range(NBUF):
            fire_didx(REM + b, b)
            fire_gather(REM + b, b)

        def step(ci, b):
            pltpu.make_async_copy(
                table.at[sidx(ci)], rows[b], gsems[b]).wait()
            pltpu.make_async_copy(
                dstw.at[wid * C + ci], dbuf[b], dsems[b]).wait()
            pltpu.sync_copy(rows[b], acc.at[dbuf[b].at[0]], add=True)

        def outer(g, carry):
            for b in range(NBUF):
                ci = REM + g * NBUF + b
                step(ci, b)
                fire_didx(ci + NBUF, b)
                fire_gather(ci + NBUF, b)
            return carry

        lax.fori_loop(0, (C - REM - NBUF) // NBUF, outer, 0)
        for b in range(NBUF):
            step(C - NBUF + b, b)

        plsc.subcore_barrier()
        pltpu.sync_copy(acc.at[pl.ds(sid * RPT, RPT)],
                        out.at[cid, pl.ds(sid * RPT, RPT)])

    return segsum


_segsum128 = _make_segsum(NHID)


@functools.partial(
    pl.kernel,
    out_type=jax.ShapeDtypeStruct((NC, 2, N), jnp.float32),
    mesh=_MESH,
    scratch_types=[
        pltpu.VMEM((C, K), jnp.int32),
        pltpu.VMEM((C, K), jnp.int32),
        pltpu.VMEM((K,), jnp.float32),
        pltpu.VMEM_SHARED((N,), jnp.float32),
        pltpu.VMEM_SHARED((N,), jnp.float32),
        pltpu.SemaphoreType.DMA,
    ],
)
def _degrees(srcw, dstw, zn, out, idxs_v, idxd_v, ones_v, dout, din, sem):
    cid = lax.axis_index("c")
    sid = lax.axis_index("s")
    wid = sid * NC + cid

    @pl.when(sid == 0)
    def _():
        pltpu.sync_copy(zn, dout)
        pltpu.sync_copy(zn, din)

    for i in range(K // 16):
        ones_v[pl.ds(i * 16, 16)] = jnp.ones((16,), jnp.float32)
    pltpu.sync_copy(srcw.at[wid], idxs_v)
    pltpu.sync_copy(dstw.at[wid], idxd_v)
    plsc.subcore_barrier()

    def chunk(ci, carry):
        pltpu.sync_copy(ones_v, dout.at[idxs_v.at[ci]], add=True)
        pltpu.sync_copy(ones_v, din.at[idxd_v.at[ci]], add=True)
        return carry

    lax.fori_loop(0, C, chunk, 0)
    plsc.subcore_barrier()

    @pl.when(sid == 0)
    def _():
        pltpu.sync_copy(dout, out.at[cid, 0])
        pltpu.sync_copy(din, out.at[cid, 1])


_R = 1000


def _inv(dr, j):
    return lax.rsqrt(jnp.maximum(dr[0, j] + dr[1, j], 1.0))


def _tc_first(x, W, degs):
    def body(xr, wr, dr, orf):
        orf[...] = jnp.dot(xr[...], wr[...],
                           preferred_element_type=jnp.float32) * _inv(dr, 0)

    return pl.pallas_call(
        body,
        grid=(N // _R,),
        in_specs=[
            pl.BlockSpec((_R, x.shape[1]), lambda i: (i, 0)),
            pl.BlockSpec(W.shape, lambda i: (0, 0)),
            pl.BlockSpec((NC, 2, _R, 1), lambda i: (0, 0, i, 0)),
        ],
        out_specs=pl.BlockSpec((_R, W.shape[1]), lambda i: (i, 0)),
        out_shape=jax.ShapeDtypeStruct((N, W.shape[1]), jnp.float32),
    )(x, W, degs)


def _tc_mid(s, degs, b, W):
    D = b.shape[1]

    def body(p0r, p1r, dr, br, wr, orf):
        h = (p0r[0] + p1r[0]) * _inv(dr, 1) + br[...]
        orf[...] = jnp.dot(h, wr[...],
                           preferred_element_type=jnp.float32) * _inv(dr, 0)

    return pl.pallas_call(
        body,
        grid=(N // _R,),
        in_specs=[
            pl.BlockSpec((1, _R, D), lambda i: (0, i, 0)),
            pl.BlockSpec((1, _R, D), lambda i: (1, i, 0)),
            pl.BlockSpec((NC, 2, _R, 1), lambda i: (0, 0, i, 0)),
            pl.BlockSpec((1, D), lambda i: (0, 0)),
            pl.BlockSpec(W.shape, lambda i: (0, 0)),
        ],
        out_specs=pl.BlockSpec((_R, W.shape[1]), lambda i: (i, 0)),
        out_shape=jax.ShapeDtypeStruct((N, W.shape[1]), jnp.float32),
    )(s, s, degs, b, W)


def _tc_last(s, degs, b):
    D = b.shape[1]

    def body(p0r, p1r, dr, br, orf):
        h = ((p0r[0, :, :NCLASS] + p1r[0, :, :NCLASS]) * _inv(dr, 1)
             + br[...])
        m = jnp.max(h, axis=1, keepdims=True)
        e = jnp.exp(h - m)
        orf[...] = e / jnp.sum(e, axis=1, keepdims=True)

    return pl.pallas_call(
        body,
        grid=(N // _R,),
        in_specs=[
            pl.BlockSpec((1, _R, NHID), lambda i: (0, i, 0)),
            pl.BlockSpec((1, _R, NHID), lambda i: (1, i, 0)),
            pl.BlockSpec((NC, 2, _R, 1), lambda i: (0, 0, i, 0)),
            pl.BlockSpec((1, D), lambda i: (0, 0)),
        ],
        out_specs=pl.BlockSpec((_R, D), lambda i: (i, 0)),
        out_shape=jax.ShapeDtypeStruct((N, D), jnp.float32),
    )(s, s, degs, b)


def kernel(x, edge_index, W1, b1, W2, b2, W3, b3):
    srcf = edge_index[0].reshape(NW, C * K)
    src = edge_index[0].reshape(NW, C, K)
    dst = edge_index[1].reshape(NW, C, K)
    dstf = edge_index[1].reshape(NW * C, 1, K)

    zn = jnp.zeros((N,), jnp.float32)
    z128 = jnp.zeros((RPT, NHID), jnp.float32)
    W3p = jnp.concatenate(
        [W3, jnp.zeros((NHID, NHID - NCLASS), jnp.float32)], axis=1)

    degs = _degrees(src, dst, zn).reshape(NC, 2, N, 1)

    b1r = b1.reshape(1, NHID)
    b2r = b2.reshape(1, NHID)
    b3r = b3.reshape(1, NCLASS)

    v1 = _tc_first(x, W1, degs)
    s1 = _segsum128(v1, srcf, dstf, z128)
    v2 = _tc_mid(s1, degs, b1r, W2)
    s2 = _segsum128(v2, srcf, dstf, z128)
    v3 = _tc_mid(s2, degs, b2r, W3p)
    s3 = _segsum128(v3, srcf, dstf, z128)
    return _tc_last(s3, degs, b3r)


# speedup vs baseline: 1.0685x; 1.0685x over previous
import functools

import jax
import jax.numpy as jnp
from jax import lax
from jax.experimental import pallas as pl
from jax.experimental.pallas import tpu as pltpu
from jax.experimental.pallas import tpu_sc as plsc

N = 10000
E = 320000
NFEAT = 128
NHID = 128
NCLASS = 64

NC, NS = 2, 16
NW = NC * NS
EPW = E // NW
K = 80
C = EPW // K
NACC = 10240
RPT = NACC // NS

_MESH = plsc.VectorSubcoreMesh(
    core_axis_name="c", subcore_axis_name="s", num_cores=NC, num_subcores=NS)


NBUF = 3
REM = (C - NBUF) % NBUF


def _make_segsum(D):
    @functools.partial(
        pl.kernel,
        out_type=jax.ShapeDtypeStruct((NC, NACC, D), jnp.float32),
        mesh=_MESH,
        scratch_types=[
            pltpu.VMEM((C * K,), jnp.int32),
        ] + [pltpu.VMEM((K, D), jnp.float32)] * NBUF
          + [pltpu.VMEM((1, K), jnp.int32)] * NBUF + [
            pltpu.VMEM_SHARED((NACC, D), jnp.float32),
        ] + [pltpu.SemaphoreType.DMA] * (2 * NBUF),
    )
    def segsum(table, srcw, dstw, zrows, out, idxs_v, *rest):
        rows = rest[:NBUF]
        dbuf = rest[NBUF:2 * NBUF]
        acc = rest[2 * NBUF]
        gsems = rest[2 * NBUF + 1:3 * NBUF + 1]
        dsems = rest[3 * NBUF + 1:]
        cid = lax.axis_index("c")
        sid = lax.axis_index("s")
        wid = sid * NC + cid
        pltpu.sync_copy(zrows, acc.at[pl.ds(sid * RPT, RPT)])
        pltpu.sync_copy(srcw.at[wid], idxs_v)
        plsc.subcore_barrier()

        def sidx(ci):
            return idxs_v.at[pl.ds(ci * K, K)]

        def fire_gather(ci, b):
            pltpu.async_copy(table.at[sidx(ci)], rows[b], gsems[b])

        def fire_didx(ci, b):
            pltpu.async_copy(dstw.at[wid * C + ci], dbuf[b], dsems[b])

        for ci in range(REM):
            pltpu.async_copy(dstw.at[wid * C + ci], dbuf[0], dsems[0])
            pltpu.async_copy(table.at[sidx(ci)], rows[0], gsems[0]).wait()
            pltpu.make_async_copy(
                dstw.at[wid * C + ci], dbuf[0], dsems[0]).wait()
            pltpu.sync_copy(rows[0], acc.at[dbuf[0].at[0]], add=True)

        for b in range(NBUF):
            fire_didx(REM + b, b)
            fire_gather(REM + b, b)

        def step(ci, b):
            pltpu.make_async_copy(
                table.at[sidx(ci)], rows[b], gsems[b]).wait()
            pltpu.make_async_copy(
                dstw.at[wid * C + ci], dbuf[b], dsems[b]).wait()
            pltpu.sync_copy(rows[b], acc.at[dbuf[b].at[0]], add=True)

        def outer(g, carry):
            for b in range(NBUF):
                ci = REM + g * NBUF + b
                step(ci, b)
                fire_didx(ci + NBUF, b)
                fire_gather(ci + NBUF, b)
            return carry

        lax.fori_loop(0, (C - REM - NBUF) // NBUF, outer, 0)
        for b in range(NBUF):
            step(C - NBUF + b, b)

        plsc.subcore_barrier()
        pltpu.sync_copy(acc.at[pl.ds(sid * RPT, RPT)],
                        out.at[cid, pl.ds(sid * RPT, RPT)])

    return segsum


_segsum128 = _make_segsum(NHID)


@functools.partial(
    pl.kernel,
    out_type=jax.ShapeDtypeStruct((NC, 2, N), jnp.float32),
    mesh=_MESH,
    scratch_types=[
        pltpu.VMEM((C, K), jnp.int32),
        pltpu.VMEM((C, K), jnp.int32),
        pltpu.VMEM((K,), jnp.float32),
        pltpu.VMEM_SHARED((N,), jnp.float32),
        pltpu.VMEM_SHARED((N,), jnp.float32),
        pltpu.SemaphoreType.DMA,
        pltpu.SemaphoreType.DMA,
    ],
)
def _degrees(srcw, dstw, zn, out, idxs_v, idxd_v, ones_v, dout, din, sem,
             sem2):
    cid = lax.axis_index("c")
    sid = lax.axis_index("s")
    wid = sid * NC + cid

    @pl.when(sid == 0)
    def _():
        pltpu.sync_copy(zn, dout)
        pltpu.sync_copy(zn, din)

    for i in range(K // 16):
        ones_v[pl.ds(i * 16, 16)] = jnp.ones((16,), jnp.float32)
    pltpu.sync_copy(srcw.at[wid], idxs_v)
    pltpu.sync_copy(dstw.at[wid], idxd_v)
    plsc.subcore_barrier()

    def chunk(ci, carry):
        pltpu.async_copy(ones_v, dout.at[idxs_v.at[ci]], sem, add=True)
        pltpu.async_copy(ones_v, din.at[idxd_v.at[ci]], sem2, add=True)
        return carry

    def drain(ci, carry):
        pltpu.make_async_copy(ones_v, dout.at[idxs_v.at[0]], sem).wait()
        pltpu.make_async_copy(ones_v, din.at[idxd_v.at[0]], sem2).wait()
        return carry

    lax.fori_loop(0, C, chunk, 0)
    lax.fori_loop(0, C, drain, 0)
    plsc.subcore_barrier()

    @pl.when(sid == 0)
    def _():
        pltpu.sync_copy(dout, out.at[cid, 0])
        pltpu.sync_copy(din, out.at[cid, 1])


def _inv_scales(deg_parts):
    def body(dr, orf):
        d = dr[0] + dr[1]
        orf[...] = lax.rsqrt(jnp.maximum(d, 1.0))

    return pl.pallas_call(
        body,
        out_shape=jax.ShapeDtypeStruct((2, N), jnp.float32),
    )(deg_parts)


_R = 1000


def _tc_first(x, W, so):
    def body(xr, wr, sr, orf):
        orf[...] = jnp.dot(xr[...], wr[...],
                           preferred_element_type=jnp.float32) * sr[...]

    return pl.pallas_call(
        body,
        grid=(N // _R,),
        in_specs=[
            pl.BlockSpec((_R, x.shape[1]), lambda i: (i, 0)),
            pl.BlockSpec(W.shape, lambda i: (0, 0)),
            pl.BlockSpec((_R, 1), lambda i: (i, 0)),
        ],
        out_specs=pl.BlockSpec((_R, W.shape[1]), lambda i: (i, 0)),
        out_shape=jax.ShapeDtypeStruct((N, W.shape[1]), jnp.float32),
    )(x, W, so)


def _tc_mid(s, si, b, W, so):
    D = b.shape[1]

    def body(p0r, p1r, sir, br, wr, sor, orf):
        h = (p0r[0] + p1r[0]) * sir[...] + br[...]
        orf[...] = jnp.dot(h, wr[...],
                           preferred_element_type=jnp.float32) * sor[...]

    return pl.pallas_call(
        body,
        grid=(N // _R,),
        in_specs=[
            pl.BlockSpec((1, _R, D), lambda i: (0, i, 0)),
            pl.BlockSpec((1, _R, D), lambda i: (1, i, 0)),
            pl.BlockSpec((_R, 1), lambda i: (i, 0)),
            pl.BlockSpec((1, D), lambda i: (0, 0)),
            pl.BlockSpec(W.shape, lambda i: (0, 0)),
            pl.BlockSpec((_R, 1), lambda i: (i, 0)),
        ],
        out_specs=pl.BlockSpec((_R, W.shape[1]), lambda i: (i, 0)),
        out_shape=jax.ShapeDtypeStruct((N, W.shape[1]), jnp.float32),
    )(s, s, si, b, W, so)


def _tc_last(s, si, b):
    D = b.shape[1]

    def body(p0r, p1r, sir, br, orf):
        h = (p0r[0, :, :NCLASS] + p1r[0, :, :NCLASS]) * sir[...] + br[...]
        m = jnp.max(h, axis=1, keepdims=True)
        e = jnp.exp(h - m)
        orf[...] = e / jnp.sum(e, axis=1, keepdims=True)

    return pl.pallas_call(
        body,
        grid=(N // _R,),
        in_specs=[
            pl.BlockSpec((1, _R, NHID), lambda i: (0, i, 0)),
            pl.BlockSpec((1, _R, NHID), lambda i: (1, i, 0)),
            pl.BlockSpec((_R, 1), lambda i: (i, 0)),
            pl.BlockSpec((1, D), lambda i: (0, 0)),
        ],
        out_specs=pl.BlockSpec((_R, D), lambda i: (i, 0)),
        out_shape=jax.ShapeDtypeStruct((N, D), jnp.float32),
    )(s, s, si, b)


def kernel(x, edge_index, W1, b1, W2, b2, W3, b3):
    srcf = edge_index[0].reshape(NW, C * K)
    src = edge_index[0].reshape(NW, C, K)
    dst = edge_index[1].reshape(NW, C, K)
    dstf = edge_index[1].reshape(NW * C, 1, K)

    zn = jnp.zeros((N,), jnp.float32)
    z128 = jnp.zeros((RPT, NHID), jnp.float32)
    W3p = jnp.concatenate(
        [W3, jnp.zeros((NHID, NHID - NCLASS), jnp.float32)], axis=1)

    deg_parts = _degrees(src, dst, zn)
    invs = _inv_scales(deg_parts)
    inv_out = invs[0].reshape(N, 1)
    inv_in = invs[1].reshape(N, 1)

    b1r = b1.reshape(1, NHID)
    b2r = b2.reshape(1, NHID)
    b3r = b3.reshape(1, NCLASS)

    v1 = _tc_first(x, W1, inv_out)
    s1 = _segsum128(v1, srcf, dstf, z128)
    v2 = _tc_mid(s1, inv_in, b1r, W2, inv_out)
    s2 = _segsum128(v2, srcf, dstf, z128)
    v3 = _tc_mid(s2, inv_in, b2r, W3p, inv_out)
    s3 = _segsum128(v3, srcf, dstf, z128)
    return _tc_last(s3, inv_in, b3r)
